# baseline (speedup 1.0000x reference)
import jax
import jax.numpy as jnp
from jax import lax
from jax.experimental import pallas as pl
from jax.experimental.pallas import tpu as pltpu

DIM = 256
HD = 64
NH = 4
GRID = 32
N = GRID * GRID
KF = 256
F32 = jnp.float32


def _dot(a, b, ca, cb):
    return lax.dot_general(a, b, (((ca,), (cb,)), ((), ())),
                           preferred_element_type=F32)


def _dotb(a, b, ca, cb):
    return lax.dot_general(a.astype(jnp.bfloat16), b.astype(jnp.bfloat16),
                           (((ca,), (cb,)), ((), ())),
                           preferred_element_type=F32)


def _k1_body(xr_ref, wd_ref, bd_ref, out_ref):
    acc = jnp.zeros((DIM, N), F32)
    for di in range(4):
        pa = (di - 1) % 2
        si = (di - 1) // 2
        for dj in range(4):
            pb = (dj - 1) % 2
            sj = (dj - 1) // 2
            xs = xr_ref[0, pa, pb, :, 1 + si:33 + si, 1 + sj:33 + sj]
            xs = xs.reshape(DIM, N)
            acc = acc + _dot(wd_ref[di, dj], xs, 1, 0)
    out_ref[0] = acc + bd_ref[:, 0][:, None]


def _conv_down(x, W_down, b_down):
    B = x.shape[0]
    xr = x.reshape(B, DIM, GRID, 2, GRID, 2).transpose(0, 3, 5, 1, 2, 4)
    xr = jnp.pad(xr, ((0, 0), (0, 0), (0, 0), (0, 0), (1, 1), (1, 1)))
    wd = W_down.transpose(2, 3, 0, 1)
    bd = b_down.reshape(DIM, 1)
    return pl.pallas_call(
        _k1_body,
        grid=(B,),
        in_specs=[
            pl.BlockSpec((1, 2, 2, DIM, 34, 34), lambda b: (b, 0, 0, 0, 0, 0)),
            pl.BlockSpec((4, 4, DIM, DIM), lambda b: (0, 0, 0, 0)),
            pl.BlockSpec((DIM, 1), lambda b: (0, 0)),
        ],
        out_specs=pl.BlockSpec((1, DIM, N), lambda b: (b, 0, 0)),
        out_shape=jax.ShapeDtypeStruct((B, DIM, N), F32),
    )(xr, wd, bd)


def _k2_body(xt_ref, w_ref, b_ref, out_ref, ca_ref):
    xt = xt_ref[0, 0]
    qkv = _dot(xt, w_ref[...], 1, 1) + b_ref[0]
    q = qkv[:, :HD]
    k = qkv[:, HD:2 * HD]
    v = qkv[:, 2 * HD:]
    s = _dot(k, q, 1, 1)
    m = jnp.max(s, axis=1, keepdims=True)
    e = jnp.exp(s - m)
    l = jnp.sum(e, axis=1, keepdims=True)
    a = e / l
    ca_ref[0, 0] = jnp.sum(a, axis=0)
    out_ref[0, 0] = _dotb(a, v, 1, 0)


def _attn1(xt, W_qkv1, b_qkv1):
    B = xt.shape[0]
    out, ca = pl.pallas_call(
        _k2_body,
        grid=(B, NH),
        in_specs=[
            pl.BlockSpec((1, 1, N, HD), lambda b, h: (b, h, 0, 0)),
            pl.BlockSpec((3 * HD, HD), lambda b, h: (0, 0)),
            pl.BlockSpec((1, 3 * HD), lambda b, h: (0, 0)),
        ],
        out_specs=[
            pl.BlockSpec((1, 1, N, HD), lambda b, h: (b, h, 0, 0)),
            pl.BlockSpec((1, 1, N), lambda b, h: (b * NH + h, 0, 0)),
        ],
        out_shape=[
            jax.ShapeDtypeStruct((B, NH, N, HD), F32),
            jax.ShapeDtypeStruct((B * NH, 1, N), F32),
        ],
    )(xt, W_qkv1, b_qkv1.reshape(1, 3 * HD))
    return out, ca


_CT_TAPS = {0: ((1, 0), (3, -1)), 1: ((2, 0), (0, 1))}


def _k4_body(xu_ref, wu_ref, bu_ref, out_ref):
    for a in range(2):
        for b2 in range(2):
            acc = jnp.zeros((DIM, N), F32)
            for (di, si) in _CT_TAPS[a]:
                for (dj, sj) in _CT_TAPS[b2]:
                    xs = xu_ref[0, :, 1 + si:33 + si, 1 + sj:33 + sj]
                    xs = xs.reshape(DIM, N)
                    acc = acc + _dotb(wu_ref[di, dj], xs, 0, 0)
            out_ref[0, a, b2] = (acc + bu_ref[:, 0][:, None]).reshape(
                DIM, GRID, GRID)


def _conv_up(xu, W_up, b_up):
    B = xu.shape[0]
    xup = jnp.pad(xu, ((0, 0), (0, 0), (1, 1), (1, 1)))
    wu = W_up.transpose(2, 3, 0, 1)
    bu = b_up.reshape(DIM, 1)
    return pl.pallas_call(
        _k4_body,
        grid=(B,),
        in_specs=[
            pl.BlockSpec((1, DIM, 34, 34), lambda b: (b, 0, 0, 0)),
            pl.BlockSpec((4, 4, DIM, DIM), lambda b: (0, 0, 0, 0)),
            pl.BlockSpec((DIM, 1), lambda b: (0, 0)),
        ],
        out_specs=pl.BlockSpec((1, 2, 2, DIM, GRID, GRID),
                               lambda b: (b, 0, 0, 0, 0, 0)),
        out_shape=jax.ShapeDtypeStruct((B, 2, 2, DIM, GRID, GRID), F32),
    )(xup, wu, bu)


def _k5_body(ca_ref, cph_ref, w_ref, b_ref, scat_ref):
    ca = ca_ref[0]
    ca_i = lax.bitcast_convert_type(ca, jnp.int32)
    p = jnp.int32(0)
    for bit in range(30, -1, -1):
        cand = p | jnp.int32(1 << bit)
        cnt = jnp.sum((ca_i >= cand).astype(jnp.int32))
        p = jnp.where(cnt >= KF, cand, p)
    gt = (ca_i > p)
    eq = (ca_i == p)
    m = jnp.sum(gt.astype(jnp.int32))
    need = (KF - m).astype(F32)
    tri = (lax.broadcasted_iota(jnp.int32, (N, N), 0)
           <= lax.broadcasted_iota(jnp.int32, (N, N), 1)).astype(F32)
    cum_eq = _dot(eq.astype(F32), tri, 1, 0)
    sel = jnp.logical_or(gt, jnp.logical_and(eq, cum_eq <= need))
    self_f = sel.astype(F32)
    cs = _dot(self_f, tri, 1, 0)
    pos = cs - 1.0
    rows = lax.broadcasted_iota(jnp.int32, (KF, N), 0).astype(F32)
    smat = jnp.where(jnp.logical_and(rows == pos, self_f > 0.5), 1.0, 0.0)
    toks = []
    for u in range(2):
        for v in range(2):
            toks.append(_dotb(smat, cph_ref[0, 0, u, v], 1, 1))
    tok = jnp.concatenate(toks, axis=0)
    qkv = _dotb(tok, w_ref[...], 1, 1) + b_ref[0]
    q = qkv[:, :HD]
    k = qkv[:, HD:2 * HD]
    v2 = qkv[:, 2 * HD:]
    s = _dotb(k, q, 1, 1)
    mx = jnp.max(s, axis=1, keepdims=True)
    e = jnp.exp(s - mx)
    l = jnp.sum(e, axis=1, keepdims=True)
    a = e / l
    out2 = _dotb(a, v2, 1, 0)
    for t in range(4):
        u, v = t // 2, t % 2
        o = out2[KF * t:KF * (t + 1)]
        scat_ref[0, 0, u, v] = _dotb(smat, o, 0, 0)


def _topk_attn2(ca, cph, W_qkv2, b_qkv2):
    B = cph.shape[0]
    return pl.pallas_call(
        _k5_body,
        grid=(B, NH),
        in_specs=[
            pl.BlockSpec((1, 1, N), lambda b, h: (b * NH + h, 0, 0)),
            pl.BlockSpec((1, 1, 2, 2, HD, N), lambda b, h: (b, h, 0, 0, 0, 0)),
            pl.BlockSpec((3 * HD, HD), lambda b, h: (0, 0)),
            pl.BlockSpec((1, 3 * HD), lambda b, h: (0, 0)),
        ],
        out_specs=pl.BlockSpec((1, 1, 2, 2, N, HD),
                               lambda b, h: (b, h, 0, 0, 0, 0)),
        out_shape=jax.ShapeDtypeStruct((B, NH, 2, 2, N, HD), F32),
    )(ca, cph, W_qkv2, b_qkv2.reshape(1, 3 * HD))


def _k6_body(c_sp_ref, c_scr_ref, s_scr_ref, wdw_ref, wpw_ref,
             s1_ref, o1_ref, s2_ref, o2_ref, out_ref, yp_ref):
    yflat = c_sp_ref[0] + c_scr_ref[0] + s_scr_ref[0]
    yp_ref[...] = jnp.zeros((DIM, 66, 66), F32)
    for r in range(64):
        yp_ref[:, 1 + r, 1:65] = yflat[:, 64 * r:64 * r + 64]
    s1 = s1_ref[:, 0][:, None]
    o1 = o1_ref[:, 0][:, None]
    s2 = s2_ref[:, 0][:, None]
    o2 = o2_ref[:, 0][:, None]
    wpw = wpw_ref[...]

    def chunk(c, _):
        ys = yp_ref[:, pl.ds(8 * c, 10), :]
        outs = []
        for rr in range(8):
            acc = jnp.zeros((DIM, 64), F32)
            for dy in range(3):
                for dx in range(3):
                    w = wdw_ref[dy * 3 + dx, :][:, None]
                    acc = acc + w * ys[:, rr + dy, dx:dx + 64]
            t = jnp.clip(acc * s1 + o1, 0.0, 6.0)
            z = _dotb(wpw, t, 1, 0)
            outs.append(jnp.clip(z * s2 + o2, 0.0, 6.0))
        out_ref[0, :, pl.ds(512 * c, 512)] = jnp.concatenate(outs, axis=1)
        return 0

    lax.fori_loop(0, 8, chunk, 0)


def _combine(c_sp, c_scr, s_scr, W_dw, W_pw, g1, b1, m1, v1, g2, b2, m2, v2):
    B = c_sp.shape[0]
    c_sp = c_sp.reshape(B, DIM, 4096)
    c_scr = c_scr.reshape(B, DIM, 4096)
    s_scr = s_scr.reshape(B, DIM, 4096)
    inv1 = g1 / jnp.sqrt(v1 + 1e-5)
    inv2 = g2 / jnp.sqrt(v2 + 1e-5)
    s1 = inv1.reshape(DIM, 1)
    o1 = (b1 - m1 * inv1).reshape(DIM, 1)
    s2 = inv2.reshape(DIM, 1)
    o2 = (b2 - m2 * inv2).reshape(DIM, 1)
    wdw = W_dw.reshape(DIM, 9).T.reshape(9, DIM)
    wpw = W_pw.reshape(DIM, DIM)
    full = lambda shape: pl.BlockSpec(shape, lambda b: (0,) * len(shape))
    return pl.pallas_call(
        _k6_body,
        grid=(B,),
        in_specs=[
            pl.BlockSpec((1, DIM, 4096), lambda b: (b, 0, 0)),
            pl.BlockSpec((1, DIM, 4096), lambda b: (b, 0, 0)),
            pl.BlockSpec((1, DIM, 4096), lambda b: (b, 0, 0)),
            full((9, DIM)),
            full((DIM, DIM)),
            full((DIM, 1)), full((DIM, 1)), full((DIM, 1)), full((DIM, 1)),
        ],
        out_specs=pl.BlockSpec((1, DIM, 4096), lambda b: (b, 0, 0)),
        out_shape=jax.ShapeDtypeStruct((B, DIM, 4096), F32),
        scratch_shapes=[pltpu.VMEM((DIM, 66, 66), F32)],
    )(c_sp, c_scr, s_scr, wdw, wpw, s1, o1, s2, o2)


def kernel(x, W_down, b_down, W_up, b_up, W_qkv1, b_qkv1, W_qkv2, b_qkv2,
           W_dw, W_pw, gamma1, beta1, mean1, var1, gamma2, beta2, mean2,
           var2):
    B = x.shape[0]
    xd = _conv_down(x, W_down, b_down)
    xt = xd.reshape(B, NH, HD, N).transpose(0, 1, 3, 2)
    out1, ca = _attn1(xt, W_qkv1, b_qkv1)
    xu = out1.transpose(0, 1, 3, 2).reshape(B, DIM, GRID, GRID)
    cp = _conv_up(xu, W_up, b_up)
    cph = cp.reshape(B, 2, 2, NH, HD, N).transpose(0, 3, 1, 2, 4, 5)
    scat = _topk_attn2(ca, cph, W_qkv2, b_qkv2)
    c_sp = cp.transpose(0, 3, 4, 1, 5, 2).reshape(B, DIM, 64, 64)
    c_scr = cp.reshape(B, 2, 2, DIM, GRID, 2, 16).transpose(
        0, 3, 4, 5, 6, 1, 2).reshape(B, DIM, 64, 64)
    s_scr = scat.reshape(B, NH, 2, 2, GRID, 2, 16, HD).transpose(
        0, 1, 7, 4, 5, 6, 2, 3).reshape(B, DIM, 64, 64)
    y = _combine(c_sp, c_scr, s_scr, W_dw, W_pw, gamma1, beta1, mean1,
                 var1, gamma2, beta2, mean2, var2)
    return y.reshape(B, DIM, 64, 64)


# speedup vs baseline: 1.0888x; 1.0888x over previous
import jax
import jax.numpy as jnp
from jax import lax
from jax.experimental import pallas as pl
from jax.experimental.pallas import tpu as pltpu

DIM = 256
HD = 64
NH = 4
GRID = 32
N = GRID * GRID
KF = 256
F32 = jnp.float32


def _dot(a, b, ca, cb):
    return lax.dot_general(a, b, (((ca,), (cb,)), ((), ())),
                           preferred_element_type=F32)


def _dotb(a, b, ca, cb):
    return lax.dot_general(a.astype(jnp.bfloat16), b.astype(jnp.bfloat16),
                           (((ca,), (cb,)), ((), ())),
                           preferred_element_type=F32)


def _k1_body(xr_ref, wd_ref, bd_ref, out_ref):
    acc = jnp.zeros((DIM, N), F32)
    for di in range(4):
        pa = (di - 1) % 2
        si = (di - 1) // 2
        for dj in range(4):
            pb = (dj - 1) % 2
            sj = (dj - 1) // 2
            xs = xr_ref[0, pa, pb, :, 1 + si:33 + si, 1 + sj:33 + sj]
            xs = xs.reshape(DIM, N)
            acc = acc + _dot(wd_ref[di, dj], xs, 1, 0)
    out_ref[0] = acc + bd_ref[:, 0][:, None]


def _conv_down(x, W_down, b_down):
    B = x.shape[0]
    xr = x.reshape(B, DIM, GRID, 2, GRID, 2).transpose(0, 3, 5, 1, 2, 4)
    xr = jnp.pad(xr, ((0, 0), (0, 0), (0, 0), (0, 0), (1, 1), (1, 1)))
    wd = W_down.transpose(2, 3, 0, 1)
    bd = b_down.reshape(DIM, 1)
    return pl.pallas_call(
        _k1_body,
        grid=(B,),
        in_specs=[
            pl.BlockSpec((1, 2, 2, DIM, 34, 34), lambda b: (b, 0, 0, 0, 0, 0)),
            pl.BlockSpec((4, 4, DIM, DIM), lambda b: (0, 0, 0, 0)),
            pl.BlockSpec((DIM, 1), lambda b: (0, 0)),
        ],
        out_specs=pl.BlockSpec((1, DIM, N), lambda b: (b, 0, 0)),
        out_shape=jax.ShapeDtypeStruct((B, DIM, N), F32),
    )(xr, wd, bd)


def _k2_body(xt_ref, w_ref, b_ref, out_ref, ca_ref):
    xt = xt_ref[0, 0]
    qkv = _dot(xt, w_ref[...], 1, 1) + b_ref[0]
    q = qkv[:, :HD]
    k = qkv[:, HD:2 * HD]
    v = qkv[:, 2 * HD:]
    s = _dot(k, q, 1, 1)
    m = jnp.max(s, axis=1, keepdims=True)
    e = jnp.exp(s - m)
    l = jnp.sum(e, axis=1, keepdims=True)
    a = e / l
    ca_ref[0, 0] = jnp.sum(a, axis=0)
    out_ref[0, 0] = _dotb(a, v, 1, 0)


def _attn1(xt, W_qkv1, b_qkv1):
    B = xt.shape[0]
    out, ca = pl.pallas_call(
        _k2_body,
        grid=(B, NH),
        in_specs=[
            pl.BlockSpec((1, 1, N, HD), lambda b, h: (b, h, 0, 0)),
            pl.BlockSpec((3 * HD, HD), lambda b, h: (0, 0)),
            pl.BlockSpec((1, 3 * HD), lambda b, h: (0, 0)),
        ],
        out_specs=[
            pl.BlockSpec((1, 1, N, HD), lambda b, h: (b, h, 0, 0)),
            pl.BlockSpec((1, 1, N), lambda b, h: (b * NH + h, 0, 0)),
        ],
        out_shape=[
            jax.ShapeDtypeStruct((B, NH, N, HD), F32),
            jax.ShapeDtypeStruct((B * NH, 1, N), F32),
        ],
    )(xt, W_qkv1, b_qkv1.reshape(1, 3 * HD))
    return out, ca


_CT_TAPS = {0: ((1, 0), (3, -1)), 1: ((2, 0), (0, 1))}


def _k4_body(xu_ref, wu_ref, bu_ref, out_ref):
    for a in range(2):
        for b2 in range(2):
            acc = jnp.zeros((DIM, N), F32)
            for (di, si) in _CT_TAPS[a]:
                for (dj, sj) in _CT_TAPS[b2]:
                    xs = xu_ref[0, :, 1 + si:33 + si, 1 + sj:33 + sj]
                    xs = xs.reshape(DIM, N)
                    acc = acc + _dotb(wu_ref[di, dj], xs, 0, 0)
            out_ref[0, a, b2] = (acc + bu_ref[:, 0][:, None]).reshape(
                DIM, GRID, GRID)


def _conv_up(xu, W_up, b_up):
    B = xu.shape[0]
    xup = jnp.pad(xu, ((0, 0), (0, 0), (1, 1), (1, 1)))
    wu = W_up.transpose(2, 3, 0, 1)
    bu = b_up.reshape(DIM, 1)
    return pl.pallas_call(
        _k4_body,
        grid=(B,),
        in_specs=[
            pl.BlockSpec((1, DIM, 34, 34), lambda b: (b, 0, 0, 0)),
            pl.BlockSpec((4, 4, DIM, DIM), lambda b: (0, 0, 0, 0)),
            pl.BlockSpec((DIM, 1), lambda b: (0, 0)),
        ],
        out_specs=pl.BlockSpec((1, 2, 2, DIM, GRID, GRID),
                               lambda b: (b, 0, 0, 0, 0, 0)),
        out_shape=jax.ShapeDtypeStruct((B, 2, 2, DIM, GRID, GRID), F32),
    )(xup, wu, bu)


def _k3_body(ca_ref, pos_ref):
    ca_i = lax.bitcast_convert_type(ca_ref[...], jnp.int32)
    p = jnp.zeros((ca_ref.shape[0], 1), jnp.int32)
    for bit in range(30, -1, -1):
        cand = p | jnp.int32(1 << bit)
        cnt = jnp.sum((ca_i >= cand).astype(jnp.int32), axis=1, keepdims=True)
        p = jnp.where(cnt >= KF, cand, p)
    gt = (ca_i > p)
    eq = (ca_i == p)
    m = jnp.sum(gt.astype(jnp.int32), axis=1, keepdims=True)
    need = (KF - m).astype(F32)
    tri = (lax.broadcasted_iota(jnp.int32, (N, N), 0)
           <= lax.broadcasted_iota(jnp.int32, (N, N), 1)).astype(F32)
    cum_eq = _dot(eq.astype(F32), tri, 1, 0)
    sel = jnp.logical_or(gt, jnp.logical_and(eq, cum_eq <= need))
    sel_f = sel.astype(F32)
    cs = _dot(sel_f, tri, 1, 0)
    pos_ref[...] = jnp.where(sel, cs - 1.0, -1.0)


def _topk_pos(ca_all):
    R = ca_all.shape[0]
    return pl.pallas_call(
        _k3_body,
        grid=(1,),
        in_specs=[pl.BlockSpec((R, N), lambda i: (0, 0))],
        out_specs=pl.BlockSpec((R, N), lambda i: (0, 0)),
        out_shape=jax.ShapeDtypeStruct((R, N), F32),
    )(ca_all)


def _k5_body(pos_ref, cph_ref, w_ref, b_ref, scat_ref):
    posv = pos_ref[0]
    rows = lax.broadcasted_iota(jnp.int32, (KF, N), 0).astype(F32)
    smat = jnp.where(jnp.logical_and(rows == posv, posv >= 0.0), 1.0, 0.0)
    toks = []
    for u in range(2):
        for v in range(2):
            toks.append(_dotb(smat, cph_ref[0, 0, u, v], 1, 1))
    tok = jnp.concatenate(toks, axis=0)
    qkv = _dotb(tok, w_ref[...], 1, 1) + b_ref[0]
    q = qkv[:, :HD]
    k = qkv[:, HD:2 * HD]
    v2 = qkv[:, 2 * HD:]
    s = _dotb(k, q, 1, 1)
    mx = jnp.max(s, axis=1, keepdims=True)
    e = jnp.exp(s - mx)
    l = jnp.sum(e, axis=1, keepdims=True)
    a = e / l
    out2 = _dotb(a, v2, 1, 0)
    for t in range(4):
        u, v = t // 2, t % 2
        o = out2[KF * t:KF * (t + 1)]
        scat_ref[0, 0, u, v] = _dotb(smat, o, 0, 0)


def _topk_attn2(pos, cph, W_qkv2, b_qkv2):
    B = cph.shape[0]
    return pl.pallas_call(
        _k5_body,
        grid=(B, NH),
        in_specs=[
            pl.BlockSpec((1, 1, N), lambda b, h: (b * NH + h, 0, 0)),
            pl.BlockSpec((1, 1, 2, 2, HD, N), lambda b, h: (b, h, 0, 0, 0, 0)),
            pl.BlockSpec((3 * HD, HD), lambda b, h: (0, 0)),
            pl.BlockSpec((1, 3 * HD), lambda b, h: (0, 0)),
        ],
        out_specs=pl.BlockSpec((1, 1, 2, 2, N, HD),
                               lambda b, h: (b, h, 0, 0, 0, 0)),
        out_shape=jax.ShapeDtypeStruct((B, NH, 2, 2, N, HD), F32),
    )(pos, cph, W_qkv2, b_qkv2.reshape(1, 3 * HD))


def _k6_body(c_sp_ref, c_scr_ref, s_scr_ref, wdw_ref, wpw_ref,
             s1_ref, o1_ref, s2_ref, o2_ref, out_ref, yp_ref):
    yflat = c_sp_ref[0] + c_scr_ref[0] + s_scr_ref[0]
    yp_ref[...] = jnp.zeros((DIM, 66, 66), F32)
    for r in range(64):
        yp_ref[:, 1 + r, 1:65] = yflat[:, 64 * r:64 * r + 64]
    s1 = s1_ref[:, 0][:, None]
    o1 = o1_ref[:, 0][:, None]
    s2 = s2_ref[:, 0][:, None]
    o2 = o2_ref[:, 0][:, None]
    wpw = wpw_ref[...]

    def chunk(c, _):
        ys = yp_ref[:, pl.ds(8 * c, 10), :]
        outs = []
        for rr in range(8):
            acc = jnp.zeros((DIM, 64), F32)
            for dy in range(3):
                for dx in range(3):
                    w = wdw_ref[dy * 3 + dx, :][:, None]
                    acc = acc + w * ys[:, rr + dy, dx:dx + 64]
            t = jnp.clip(acc * s1 + o1, 0.0, 6.0)
            z = _dotb(wpw, t, 1, 0)
            outs.append(jnp.clip(z * s2 + o2, 0.0, 6.0))
        out_ref[0, :, pl.ds(512 * c, 512)] = jnp.concatenate(outs, axis=1)
        return 0

    lax.fori_loop(0, 8, chunk, 0)


def _combine(c_sp, c_scr, s_scr, W_dw, W_pw, g1, b1, m1, v1, g2, b2, m2, v2):
    B = c_sp.shape[0]
    c_sp = c_sp.reshape(B, DIM, 4096)
    c_scr = c_scr.reshape(B, DIM, 4096)
    s_scr = s_scr.reshape(B, DIM, 4096)
    inv1 = g1 / jnp.sqrt(v1 + 1e-5)
    inv2 = g2 / jnp.sqrt(v2 + 1e-5)
    s1 = inv1.reshape(DIM, 1)
    o1 = (b1 - m1 * inv1).reshape(DIM, 1)
    s2 = inv2.reshape(DIM, 1)
    o2 = (b2 - m2 * inv2).reshape(DIM, 1)
    wdw = W_dw.reshape(DIM, 9).T.reshape(9, DIM)
    wpw = W_pw.reshape(DIM, DIM)
    full = lambda shape: pl.BlockSpec(shape, lambda b: (0,) * len(shape))
    return pl.pallas_call(
        _k6_body,
        grid=(B,),
        in_specs=[
            pl.BlockSpec((1, DIM, 4096), lambda b: (b, 0, 0)),
            pl.BlockSpec((1, DIM, 4096), lambda b: (b, 0, 0)),
            pl.BlockSpec((1, DIM, 4096), lambda b: (b, 0, 0)),
            full((9, DIM)),
            full((DIM, DIM)),
            full((DIM, 1)), full((DIM, 1)), full((DIM, 1)), full((DIM, 1)),
        ],
        out_specs=pl.BlockSpec((1, DIM, 4096), lambda b: (b, 0, 0)),
        out_shape=jax.ShapeDtypeStruct((B, DIM, 4096), F32),
        scratch_shapes=[pltpu.VMEM((DIM, 66, 66), F32)],
    )(c_sp, c_scr, s_scr, wdw, wpw, s1, o1, s2, o2)


def kernel(x, W_down, b_down, W_up, b_up, W_qkv1, b_qkv1, W_qkv2, b_qkv2,
           W_dw, W_pw, gamma1, beta1, mean1, var1, gamma2, beta2, mean2,
           var2):
    B = x.shape[0]
    xd = _conv_down(x, W_down, b_down)
    xt = xd.reshape(B, NH, HD, N).transpose(0, 1, 3, 2)
    out1, ca = _attn1(xt, W_qkv1, b_qkv1)
    xu = out1.transpose(0, 1, 3, 2).reshape(B, DIM, GRID, GRID)
    cp = _conv_up(xu, W_up, b_up)
    cph = cp.reshape(B, 2, 2, NH, HD, N).transpose(0, 3, 1, 2, 4, 5)
    pos = _topk_pos(ca.reshape(B * NH, N)).reshape(B * NH, 1, N)
    scat = _topk_attn2(pos, cph, W_qkv2, b_qkv2)
    c_sp = cp.transpose(0, 3, 4, 1, 5, 2).reshape(B, DIM, 64, 64)
    c_scr = cp.reshape(B, 2, 2, DIM, GRID, 2, 16).transpose(
        0, 3, 4, 5, 6, 1, 2).reshape(B, DIM, 64, 64)
    s_scr = scat.reshape(B, NH, 2, 2, GRID, 2, 16, HD).transpose(
        0, 1, 7, 4, 5, 6, 2, 3).reshape(B, DIM, 64, 64)
    y = _combine(c_sp, c_scr, s_scr, W_dw, W_pw, gamma1, beta1, mean1,
                 var1, gamma2, beta2, mean2, var2)
    return y.reshape(B, DIM, 64, 64)


# speedup vs baseline: 1.1251x; 1.0334x over previous
import jax
import jax.numpy as jnp
from jax import lax
from jax.experimental import pallas as pl
from jax.experimental.pallas import tpu as pltpu

DIM = 256
HD = 64
NH = 4
GRID = 32
N = GRID * GRID
KF = 256
F32 = jnp.float32


def _dot(a, b, ca, cb):
    return lax.dot_general(a, b, (((ca,), (cb,)), ((), ())),
                           preferred_element_type=F32)


def _dotb(a, b, ca, cb):
    return lax.dot_general(a.astype(jnp.bfloat16), b.astype(jnp.bfloat16),
                           (((ca,), (cb,)), ((), ())),
                           preferred_element_type=F32)


def _k1_body(xr_ref, wd_ref, bd_ref, out_ref):
    acc = jnp.zeros((DIM, N), F32)
    for di in range(4):
        pa = (di - 1) % 2
        si = (di - 1) // 2
        for dj in range(4):
            pb = (dj - 1) % 2
            sj = (dj - 1) // 2
            xs = xr_ref[0, pa, pb, :, 1 + si:33 + si, 1 + sj:33 + sj]
            xs = xs.reshape(DIM, N)
            acc = acc + _dot(wd_ref[di, dj], xs, 1, 0)
    out_ref[0] = acc + bd_ref[:, 0][:, None]


def _conv_down(x, W_down, b_down):
    B = x.shape[0]
    xr = x.reshape(B, DIM, GRID, 2, GRID, 2).transpose(0, 3, 5, 1, 2, 4)
    xr = jnp.pad(xr, ((0, 0), (0, 0), (0, 0), (0, 0), (1, 1), (1, 1)))
    wd = W_down.transpose(2, 3, 0, 1)
    bd = b_down.reshape(DIM, 1)
    return pl.pallas_call(
        _k1_body,
        grid=(B,),
        in_specs=[
            pl.BlockSpec((1, 2, 2, DIM, 34, 34), lambda b: (b, 0, 0, 0, 0, 0)),
            pl.BlockSpec((4, 4, DIM, DIM), lambda b: (0, 0, 0, 0)),
            pl.BlockSpec((DIM, 1), lambda b: (0, 0)),
        ],
        out_specs=pl.BlockSpec((1, DIM, N), lambda b: (b, 0, 0)),
        out_shape=jax.ShapeDtypeStruct((B, DIM, N), F32),
    )(xr, wd, bd)


def _k2_body(xd_ref, w_ref, b_ref, out_ref, ca_ref):
    xh = xd_ref[0]
    qkv = _dot(w_ref[...], xh, 1, 0) + b_ref[:, 0][:, None]
    q = qkv[:HD]
    k = qkv[HD:2 * HD]
    v = qkv[2 * HD:]
    s = _dot(k, q, 0, 0)
    m = jnp.max(s, axis=1, keepdims=True)
    e = jnp.exp(s - m)
    l = jnp.sum(e, axis=1, keepdims=True)
    r = 1.0 / l
    ca_ref[0, 0] = _dot(e, r, 0, 0)[:, 0]
    g = _dotb(v, e, 1, 1)
    out_ref[0, 0] = g * r[:, 0][None, :]


def _attn1(xd, W_qkv1, b_qkv1):
    B = xd.shape[0]
    out, ca = pl.pallas_call(
        _k2_body,
        grid=(B, NH),
        in_specs=[
            pl.BlockSpec((1, HD, N), lambda b, h: (b, h, 0)),
            pl.BlockSpec((3 * HD, HD), lambda b, h: (0, 0)),
            pl.BlockSpec((3 * HD, 1), lambda b, h: (0, 0)),
        ],
        out_specs=[
            pl.BlockSpec((1, 1, HD, N), lambda b, h: (b, h, 0, 0)),
            pl.BlockSpec((1, 1, N), lambda b, h: (b * NH + h, 0, 0)),
        ],
        out_shape=[
            jax.ShapeDtypeStruct((B, NH, HD, N), F32),
            jax.ShapeDtypeStruct((B * NH, 1, N), F32),
        ],
    )(xd, W_qkv1, b_qkv1.reshape(3 * HD, 1))
    return out, ca


_CT_TAPS = {0: ((1, 0), (3, -1)), 1: ((2, 0), (0, 1))}


def _k4_body(xu_ref, wu_ref, bu_ref, out_ref):
    for a in range(2):
        for b2 in range(2):
            acc = jnp.zeros((DIM, N), F32)
            for (di, si) in _CT_TAPS[a]:
                for (dj, sj) in _CT_TAPS[b2]:
                    xs = xu_ref[0, :, 1 + si:33 + si, 1 + sj:33 + sj]
                    xs = xs.reshape(DIM, N)
                    acc = acc + _dotb(wu_ref[di, dj], xs, 0, 0)
            out_ref[0, a, b2] = (acc + bu_ref[:, 0][:, None]).reshape(
                DIM, GRID, GRID)


def _conv_up(xu, W_up, b_up):
    B = xu.shape[0]
    xup = jnp.pad(xu, ((0, 0), (0, 0), (1, 1), (1, 1)))
    wu = W_up.transpose(2, 3, 0, 1)
    bu = b_up.reshape(DIM, 1)
    return pl.pallas_call(
        _k4_body,
        grid=(B,),
        in_specs=[
            pl.BlockSpec((1, DIM, 34, 34), lambda b: (b, 0, 0, 0)),
            pl.BlockSpec((4, 4, DIM, DIM), lambda b: (0, 0, 0, 0)),
            pl.BlockSpec((DIM, 1), lambda b: (0, 0)),
        ],
        out_specs=pl.BlockSpec((1, 2, 2, DIM, GRID, GRID),
                               lambda b: (b, 0, 0, 0, 0, 0)),
        out_shape=jax.ShapeDtypeStruct((B, 2, 2, DIM, GRID, GRID), F32),
    )(xup, wu, bu)


def _k3_body(ca_ref, pos_ref):
    ca_i = lax.bitcast_convert_type(ca_ref[...], jnp.int32)
    p = jnp.zeros((ca_ref.shape[0], 1), jnp.int32)
    for bit in range(30, -1, -1):
        cand = p | jnp.int32(1 << bit)
        cnt = jnp.sum((ca_i >= cand).astype(jnp.int32), axis=1, keepdims=True)
        p = jnp.where(cnt >= KF, cand, p)
    gt = (ca_i > p)
    eq = (ca_i == p)
    m = jnp.sum(gt.astype(jnp.int32), axis=1, keepdims=True)
    need = (KF - m).astype(F32)
    tri = (lax.broadcasted_iota(jnp.int32, (N, N), 0)
           <= lax.broadcasted_iota(jnp.int32, (N, N), 1)).astype(F32)
    cum_eq = _dot(eq.astype(F32), tri, 1, 0)
    sel = jnp.logical_or(gt, jnp.logical_and(eq, cum_eq <= need))
    sel_f = sel.astype(F32)
    cs = _dot(sel_f, tri, 1, 0)
    pos_ref[...] = jnp.where(sel, cs - 1.0, -1.0)


def _topk_pos(ca_all):
    R = ca_all.shape[0]
    return pl.pallas_call(
        _k3_body,
        grid=(1,),
        in_specs=[pl.BlockSpec((R, N), lambda i: (0, 0))],
        out_specs=pl.BlockSpec((R, N), lambda i: (0, 0)),
        out_shape=jax.ShapeDtypeStruct((R, N), F32),
    )(ca_all)


def _k5_body(pos_ref, cph_ref, w_ref, b_ref, scat_ref):
    posv = pos_ref[0]
    rows = lax.broadcasted_iota(jnp.int32, (KF, N), 0).astype(F32)
    smat = jnp.where(jnp.logical_and(rows == posv, posv >= 0.0), 1.0, 0.0)
    toks = []
    for u in range(2):
        for v in range(2):
            plane = cph_ref[0, u, v].reshape(HD, N)
            toks.append(_dotb(smat, plane, 1, 1))
    tok = jnp.concatenate(toks, axis=0)
    qkv = _dotb(tok, w_ref[...], 1, 1) + b_ref[0]
    q = qkv[:, :HD]
    k = qkv[:, HD:2 * HD]
    v2 = qkv[:, 2 * HD:]
    s = _dotb(k, q, 1, 1)
    mx = jnp.max(s, axis=1, keepdims=True)
    e = jnp.exp(s - mx)
    l = jnp.sum(e, axis=1, keepdims=True)
    out2 = _dotb(e, v2, 1, 0) / l
    for t in range(4):
        u, v = t // 2, t % 2
        o = out2[KF * t:KF * (t + 1)]
        scat_ref[0, 0, u, v] = _dotb(smat, o, 0, 0)


def _topk_attn2(pos, cp, W_qkv2, b_qkv2):
    B = cp.shape[0]
    return pl.pallas_call(
        _k5_body,
        grid=(B, NH),
        in_specs=[
            pl.BlockSpec((1, 1, N), lambda b, h: (b * NH + h, 0, 0)),
            pl.BlockSpec((1, 2, 2, HD, GRID, GRID),
                         lambda b, h: (b, 0, 0, h, 0, 0)),
            pl.BlockSpec((3 * HD, HD), lambda b, h: (0, 0)),
            pl.BlockSpec((1, 3 * HD), lambda b, h: (0, 0)),
        ],
        out_specs=pl.BlockSpec((1, 1, 2, 2, N, HD),
                               lambda b, h: (b, h, 0, 0, 0, 0)),
        out_shape=jax.ShapeDtypeStruct((B, NH, 2, 2, N, HD), F32),
    )(pos, cp, W_qkv2, b_qkv2.reshape(1, 3 * HD))


def _k6_body(c_sp_ref, c_scr_ref, s_scr_ref, wdw_ref, wpw_ref,
             s1_ref, o1_ref, s2_ref, o2_ref, out_ref, yp_ref):
    yflat = c_sp_ref[0] + c_scr_ref[0] + s_scr_ref[0]
    yp_ref[...] = jnp.zeros((DIM, 66, 66), F32)
    for r in range(64):
        yp_ref[:, 1 + r, 1:65] = yflat[:, 64 * r:64 * r + 64]
    s1 = s1_ref[:, 0][:, None]
    o1 = o1_ref[:, 0][:, None]
    s2 = s2_ref[:, 0][:, None]
    o2 = o2_ref[:, 0][:, None]
    wpw = wpw_ref[...]

    def chunk(c, _):
        ys = yp_ref[:, pl.ds(8 * c, 10), :]
        outs = []
        for rr in range(8):
            acc = jnp.zeros((DIM, 64), F32)
            for dy in range(3):
                for dx in range(3):
                    w = wdw_ref[dy * 3 + dx, :][:, None]
                    acc = acc + w * ys[:, rr + dy, dx:dx + 64]
            t = jnp.clip(acc * s1 + o1, 0.0, 6.0)
            z = _dotb(wpw, t, 1, 0)
            outs.append(jnp.clip(z * s2 + o2, 0.0, 6.0))
        out_ref[0, :, pl.ds(512 * c, 512)] = jnp.concatenate(outs, axis=1)
        return 0

    lax.fori_loop(0, 8, chunk, 0)


def _combine(c_sp, c_scr, s_scr, W_dw, W_pw, g1, b1, m1, v1, g2, b2, m2, v2):
    B = c_sp.shape[0]
    c_sp = c_sp.reshape(B, DIM, 4096)
    c_scr = c_scr.reshape(B, DIM, 4096)
    s_scr = s_scr.reshape(B, DIM, 4096)
    inv1 = g1 / jnp.sqrt(v1 + 1e-5)
    inv2 = g2 / jnp.sqrt(v2 + 1e-5)
    s1 = inv1.reshape(DIM, 1)
    o1 = (b1 - m1 * inv1).reshape(DIM, 1)
    s2 = inv2.reshape(DIM, 1)
    o2 = (b2 - m2 * inv2).reshape(DIM, 1)
    wdw = W_dw.reshape(DIM, 9).T.reshape(9, DIM)
    wpw = W_pw.reshape(DIM, DIM)
    full = lambda shape: pl.BlockSpec(shape, lambda b: (0,) * len(shape))
    return pl.pallas_call(
        _k6_body,
        grid=(B,),
        in_specs=[
            pl.BlockSpec((1, DIM, 4096), lambda b: (b, 0, 0)),
            pl.BlockSpec((1, DIM, 4096), lambda b: (b, 0, 0)),
            pl.BlockSpec((1, DIM, 4096), lambda b: (b, 0, 0)),
            full((9, DIM)),
            full((DIM, DIM)),
            full((DIM, 1)), full((DIM, 1)), full((DIM, 1)), full((DIM, 1)),
        ],
        out_specs=pl.BlockSpec((1, DIM, 4096), lambda b: (b, 0, 0)),
        out_shape=jax.ShapeDtypeStruct((B, DIM, 4096), F32),
        scratch_shapes=[pltpu.VMEM((DIM, 66, 66), F32)],
    )(c_sp, c_scr, s_scr, wdw, wpw, s1, o1, s2, o2)


def kernel(x, W_down, b_down, W_up, b_up, W_qkv1, b_qkv1, W_qkv2, b_qkv2,
           W_dw, W_pw, gamma1, beta1, mean1, var1, gamma2, beta2, mean2,
           var2):
    B = x.shape[0]
    xd = _conv_down(x, W_down, b_down)
    out1, ca = _attn1(xd, W_qkv1, b_qkv1)
    xu = out1.reshape(B, DIM, GRID, GRID)
    cp = _conv_up(xu, W_up, b_up)
    pos = _topk_pos(ca.reshape(B * NH, N)).reshape(B * NH, 1, N)
    scat = _topk_attn2(pos, cp, W_qkv2, b_qkv2)
    c_sp = cp.transpose(0, 3, 4, 1, 5, 2).reshape(B, DIM, 64, 64)
    c_scr = cp.reshape(B, 2, 2, DIM, GRID, 2, 16).transpose(
        0, 3, 4, 5, 6, 1, 2).reshape(B, DIM, 64, 64)
    s_scr = scat.reshape(B, NH, 2, 2, GRID, 2, 16, HD).transpose(
        0, 1, 7, 4, 5, 6, 2, 3).reshape(B, DIM, 64, 64)
    y = _combine(c_sp, c_scr, s_scr, W_dw, W_pw, gamma1, beta1, mean1,
                 var1, gamma2, beta2, mean2, var2)
    return y.reshape(B, DIM, 64, 64)


# speedup vs baseline: 1.3065x; 1.1613x over previous
import jax
import jax.numpy as jnp
from jax import lax
from jax.experimental import pallas as pl
from jax.experimental.pallas import tpu as pltpu

DIM = 256
HD = 64
NH = 4
GRID = 32
N = GRID * GRID
KF = 256
F32 = jnp.float32


def _dot(a, b, ca, cb):
    return lax.dot_general(a, b, (((ca,), (cb,)), ((), ())),
                           preferred_element_type=F32)


def _dotb(a, b, ca, cb):
    return lax.dot_general(a.astype(jnp.bfloat16), b.astype(jnp.bfloat16),
                           (((ca,), (cb,)), ((), ())),
                           preferred_element_type=F32)


def _k1_body(xr_ref, wd_ref, bd_ref, out_ref):
    acc = jnp.zeros((DIM, N), F32)
    for di in range(4):
        pa = (di - 1) % 2
        si = (di - 1) // 2
        for dj in range(4):
            pb = (dj - 1) % 2
            sj = (dj - 1) // 2
            xs = xr_ref[0, pa, pb, :, 1 + si:33 + si, 1 + sj:33 + sj]
            xs = xs.reshape(DIM, N)
            acc = acc + _dot(wd_ref[di, dj], xs, 1, 0)
    out_ref[0] = acc + bd_ref[:, 0][:, None]


def _conv_down(x, W_down, b_down):
    B = x.shape[0]
    xr = x.reshape(B, DIM, GRID, 2, GRID, 2).transpose(0, 3, 5, 1, 2, 4)
    xr = jnp.pad(xr, ((0, 0), (0, 0), (0, 0), (0, 0), (1, 1), (1, 1)))
    wd = W_down.transpose(2, 3, 0, 1)
    bd = b_down.reshape(DIM, 1)
    return pl.pallas_call(
        _k1_body,
        grid=(B,),
        in_specs=[
            pl.BlockSpec((1, 2, 2, DIM, 34, 34), lambda b: (b, 0, 0, 0, 0, 0)),
            pl.BlockSpec((4, 4, DIM, DIM), lambda b: (0, 0, 0, 0)),
            pl.BlockSpec((DIM, 1), lambda b: (0, 0)),
        ],
        out_specs=pl.BlockSpec((1, DIM, N), lambda b: (b, 0, 0)),
        out_shape=jax.ShapeDtypeStruct((B, DIM, N), F32),
    )(xr, wd, bd)


def _k2_body(xd_ref, w_ref, b_ref, out_ref, ca_ref):
    xh = xd_ref[0]
    qkv = _dot(w_ref[...], xh, 1, 0) + b_ref[:, 0][:, None]
    q = qkv[:HD]
    k = qkv[HD:2 * HD]
    v = qkv[2 * HD:]
    s = _dot(k, q, 0, 0)
    m = jnp.max(s, axis=1, keepdims=True)
    e = jnp.exp(s - m)
    l = jnp.sum(e, axis=1, keepdims=True)
    r = 1.0 / l
    ca_ref[0, 0] = _dot(e, r, 0, 0)[:, 0]
    g = _dotb(v, e, 1, 1)
    out_ref[0, 0] = g * r[:, 0][None, :]


def _attn1(xd, W_qkv1, b_qkv1):
    B = xd.shape[0]
    out, ca = pl.pallas_call(
        _k2_body,
        grid=(B, NH),
        in_specs=[
            pl.BlockSpec((1, HD, N), lambda b, h: (b, h, 0)),
            pl.BlockSpec((3 * HD, HD), lambda b, h: (0, 0)),
            pl.BlockSpec((3 * HD, 1), lambda b, h: (0, 0)),
        ],
        out_specs=[
            pl.BlockSpec((1, 1, HD, N), lambda b, h: (b, h, 0, 0)),
            pl.BlockSpec((1, 1, N), lambda b, h: (b * NH + h, 0, 0)),
        ],
        out_shape=[
            jax.ShapeDtypeStruct((B, NH, HD, N), F32),
            jax.ShapeDtypeStruct((B * NH, 1, N), F32),
        ],
    )(xd, W_qkv1, b_qkv1.reshape(3 * HD, 1))
    return out, ca


_CT_TAPS = {0: ((1, 0), (3, -1)), 1: ((2, 0), (0, 1))}


def _k4_body(xu_ref, wu_ref, bu_ref, out_ref):
    for a in range(2):
        for b2 in range(2):
            acc = jnp.zeros((DIM, N), F32)
            for (di, si) in _CT_TAPS[a]:
                for (dj, sj) in _CT_TAPS[b2]:
                    xs = xu_ref[0, :, 1 + si:33 + si, 1 + sj:33 + sj]
                    xs = xs.reshape(DIM, N)
                    acc = acc + _dotb(wu_ref[di, dj], xs, 0, 0)
            out_ref[0, a, b2] = (acc + bu_ref[:, 0][:, None]).reshape(
                DIM, GRID, GRID)


def _conv_up(xu, W_up, b_up):
    B = xu.shape[0]
    xup = jnp.pad(xu, ((0, 0), (0, 0), (1, 1), (1, 1)))
    wu = W_up.transpose(2, 3, 0, 1)
    bu = b_up.reshape(DIM, 1)
    return pl.pallas_call(
        _k4_body,
        grid=(B,),
        in_specs=[
            pl.BlockSpec((1, DIM, 34, 34), lambda b: (b, 0, 0, 0)),
            pl.BlockSpec((4, 4, DIM, DIM), lambda b: (0, 0, 0, 0)),
            pl.BlockSpec((DIM, 1), lambda b: (0, 0)),
        ],
        out_specs=pl.BlockSpec((1, 2, 2, DIM, GRID, GRID),
                               lambda b: (b, 0, 0, 0, 0, 0)),
        out_shape=jax.ShapeDtypeStruct((B, 2, 2, DIM, GRID, GRID), F32),
    )(xup, wu, bu)


def _k3_body(ca_ref, pos_ref):
    ca_i = lax.bitcast_convert_type(ca_ref[...], jnp.int32)
    p = jnp.zeros((ca_ref.shape[0], 1), jnp.int32)
    for bit in range(30, -1, -1):
        cand = p | jnp.int32(1 << bit)
        cnt = jnp.sum((ca_i >= cand).astype(jnp.int32), axis=1, keepdims=True)
        p = jnp.where(cnt >= KF, cand, p)
    gt = (ca_i > p)
    eq = (ca_i == p)
    m = jnp.sum(gt.astype(jnp.int32), axis=1, keepdims=True)
    need = (KF - m).astype(F32)
    tri = (lax.broadcasted_iota(jnp.int32, (N, N), 0)
           <= lax.broadcasted_iota(jnp.int32, (N, N), 1)).astype(F32)
    cum_eq = _dot(eq.astype(F32), tri, 1, 0)
    sel = jnp.logical_or(gt, jnp.logical_and(eq, cum_eq <= need))
    sel_f = sel.astype(F32)
    cs = _dot(sel_f, tri, 1, 0)
    pos_ref[...] = jnp.where(sel, cs - 1.0, -1.0)


def _topk_pos(ca_all):
    R = ca_all.shape[0]
    return pl.pallas_call(
        _k3_body,
        grid=(1,),
        in_specs=[pl.BlockSpec((R, N), lambda i: (0, 0))],
        out_specs=pl.BlockSpec((R, N), lambda i: (0, 0)),
        out_shape=jax.ShapeDtypeStruct((R, N), F32),
    )(ca_all)


def _k5_body(pos_ref, cph_ref, w_ref, b_ref, scat_ref):
    posv = pos_ref[0]
    rows = lax.broadcasted_iota(jnp.int32, (KF, N), 0).astype(F32)
    smat = jnp.where(jnp.logical_and(rows == posv, posv >= 0.0), 1.0, 0.0)
    toks = []
    for u in range(2):
        for v in range(2):
            plane = cph_ref[0, u, v].reshape(HD, N)
            toks.append(_dotb(smat, plane, 1, 1))
    tok = jnp.concatenate(toks, axis=0)
    qkv = _dotb(tok, w_ref[...], 1, 1) + b_ref[0]
    q = qkv[:, :HD]
    k = qkv[:, HD:2 * HD]
    v2 = qkv[:, 2 * HD:]
    s = _dotb(k, q, 1, 1)
    mx = jnp.max(s, axis=1, keepdims=True)
    e = jnp.exp(s - mx)
    l = jnp.sum(e, axis=1, keepdims=True)
    out2 = _dotb(e, v2, 1, 0) / l
    for t in range(4):
        u, v = t // 2, t % 2
        o = out2[KF * t:KF * (t + 1)]
        scat_ref[0, 0, u, v] = _dotb(smat, o, 0, 0)


def _topk_attn2(pos, cp, W_qkv2, b_qkv2):
    B = cp.shape[0]
    return pl.pallas_call(
        _k5_body,
        grid=(B, NH),
        in_specs=[
            pl.BlockSpec((1, 1, N), lambda b, h: (b * NH + h, 0, 0)),
            pl.BlockSpec((1, 2, 2, HD, GRID, GRID),
                         lambda b, h: (b, 0, 0, h, 0, 0)),
            pl.BlockSpec((3 * HD, HD), lambda b, h: (0, 0)),
            pl.BlockSpec((1, 3 * HD), lambda b, h: (0, 0)),
        ],
        out_specs=pl.BlockSpec((1, 1, 2, 2, N, HD),
                               lambda b, h: (b, h, 0, 0, 0, 0)),
        out_shape=jax.ShapeDtypeStruct((B, NH, 2, 2, N, HD), F32),
    )(pos, cp, W_qkv2, b_qkv2.reshape(1, 3 * HD))


def _shift_cols(x, k):
    if k > 0:
        return jnp.concatenate(
            [x[:, k:], jnp.zeros((x.shape[0], k), F32)], axis=1)
    if k < 0:
        return jnp.concatenate(
            [jnp.zeros((x.shape[0], -k), F32), x[:, :k]], axis=1)
    return x


def _k6_body(c_sp_ref, c_scr_ref, s_scr_ref, wdw_ref, wpw_ref,
             s1_ref, o1_ref, s2_ref, o2_ref, out_ref):
    y = c_sp_ref[0] + c_scr_ref[0] + s_scr_ref[0]
    colv = lax.broadcasted_iota(jnp.int32, (1, 4096), 1) % 64
    acc = jnp.zeros((DIM, 4096), F32)
    for dy in range(3):
        for dx in range(3):
            t = _shift_cols(y, 64 * (dy - 1) + (dx - 1))
            if dx == 0:
                t = jnp.where(colv >= 1, t, 0.0)
            elif dx == 2:
                t = jnp.where(colv <= 62, t, 0.0)
            acc = acc + wdw_ref[dy * 3 + dx, :][:, None] * t
    t = jnp.clip(acc * s1_ref[:, 0][:, None] + o1_ref[:, 0][:, None],
                 0.0, 6.0)
    z = _dotb(wpw_ref[...], t, 1, 0)
    out_ref[0] = jnp.clip(z * s2_ref[:, 0][:, None] + o2_ref[:, 0][:, None],
                          0.0, 6.0)


def _combine(c_sp, c_scr, s_scr, W_dw, W_pw, g1, b1, m1, v1, g2, b2, m2, v2):
    B = c_sp.shape[0]
    c_sp = c_sp.reshape(B, DIM, 4096)
    c_scr = c_scr.reshape(B, DIM, 4096)
    s_scr = s_scr.reshape(B, DIM, 4096)
    inv1 = g1 / jnp.sqrt(v1 + 1e-5)
    inv2 = g2 / jnp.sqrt(v2 + 1e-5)
    s1 = inv1.reshape(DIM, 1)
    o1 = (b1 - m1 * inv1).reshape(DIM, 1)
    s2 = inv2.reshape(DIM, 1)
    o2 = (b2 - m2 * inv2).reshape(DIM, 1)
    wdw = W_dw.reshape(DIM, 9).T.reshape(9, DIM)
    wpw = W_pw.reshape(DIM, DIM)
    full = lambda shape: pl.BlockSpec(shape, lambda b: (0,) * len(shape))
    return pl.pallas_call(
        _k6_body,
        grid=(B,),
        in_specs=[
            pl.BlockSpec((1, DIM, 4096), lambda b: (b, 0, 0)),
            pl.BlockSpec((1, DIM, 4096), lambda b: (b, 0, 0)),
            pl.BlockSpec((1, DIM, 4096), lambda b: (b, 0, 0)),
            full((9, DIM)),
            full((DIM, DIM)),
            full((DIM, 1)), full((DIM, 1)), full((DIM, 1)), full((DIM, 1)),
        ],
        out_specs=pl.BlockSpec((1, DIM, 4096), lambda b: (b, 0, 0)),
        out_shape=jax.ShapeDtypeStruct((B, DIM, 4096), F32),
    )(c_sp, c_scr, s_scr, wdw, wpw, s1, o1, s2, o2)


def kernel(x, W_down, b_down, W_up, b_up, W_qkv1, b_qkv1, W_qkv2, b_qkv2,
           W_dw, W_pw, gamma1, beta1, mean1, var1, gamma2, beta2, mean2,
           var2):
    B = x.shape[0]
    xd = _conv_down(x, W_down, b_down)
    out1, ca = _attn1(xd, W_qkv1, b_qkv1)
    xu = out1.reshape(B, DIM, GRID, GRID)
    cp = _conv_up(xu, W_up, b_up)
    pos = _topk_pos(ca.reshape(B * NH, N)).reshape(B * NH, 1, N)
    scat = _topk_attn2(pos, cp, W_qkv2, b_qkv2)
    c_sp = cp.transpose(0, 3, 4, 1, 5, 2).reshape(B, DIM, 64, 64)
    c_scr = cp.reshape(B, 2, 2, DIM, GRID, 2, 16).transpose(
        0, 3, 4, 5, 6, 1, 2).reshape(B, DIM, 64, 64)
    s_scr = scat.reshape(B, NH, 2, 2, GRID, 2, 16, HD).transpose(
        0, 1, 7, 4, 5, 6, 2, 3).reshape(B, DIM, 64, 64)
    y = _combine(c_sp, c_scr, s_scr, W_dw, W_pw, gamma1, beta1, mean1,
                 var1, gamma2, beta2, mean2, var2)
    return y.reshape(B, DIM, 64, 64)


# speedup vs baseline: 2.0239x; 1.5490x over previous
import jax
import jax.numpy as jnp
from jax import lax
from jax.experimental import pallas as pl
from jax.experimental.pallas import tpu as pltpu

DIM = 256
HD = 64
NH = 4
GRID = 32
N = GRID * GRID
KF = 256
F32 = jnp.float32


def _dot(a, b, ca, cb):
    return lax.dot_general(a, b, (((ca,), (cb,)), ((), ())),
                           preferred_element_type=F32)


def _dotb(a, b, ca, cb):
    return lax.dot_general(a.astype(jnp.bfloat16), b.astype(jnp.bfloat16),
                           (((ca,), (cb,)), ((), ())),
                           preferred_element_type=F32)


def _shift_grid(x, si, sj):
    t = _shift_cols(x, GRID * si + sj)
    colv = lax.broadcasted_iota(jnp.int32, (1, N), 1) % GRID
    if sj == -1:
        t = jnp.where(colv >= 1, t, 0.0)
    elif sj == 1:
        t = jnp.where(colv <= GRID - 2, t, 0.0)
    return t


def _k1_body(xr_ref, wd_ref, bd_ref, out_ref):
    acc = jnp.zeros((DIM, N), F32)
    for di in range(4):
        pa = (di - 1) % 2
        si = (di - 1) // 2
        for dj in range(4):
            pb = (dj - 1) % 2
            sj = (dj - 1) // 2
            xs = _shift_grid(xr_ref[0, pa, pb], si, sj)
            acc = acc + _dot(wd_ref[di, dj], xs, 1, 0)
    out_ref[0] = acc + bd_ref[:, 0][:, None]


def _conv_down(x, W_down, b_down):
    B = x.shape[0]
    xr = x.reshape(B, DIM, GRID, 2, GRID, 2).transpose(
        0, 3, 5, 1, 2, 4).reshape(B, 2, 2, DIM, N)
    wd = W_down.transpose(2, 3, 0, 1)
    bd = b_down.reshape(DIM, 1)
    return pl.pallas_call(
        _k1_body,
        grid=(B,),
        in_specs=[
            pl.BlockSpec((1, 2, 2, DIM, N), lambda b: (b, 0, 0, 0, 0)),
            pl.BlockSpec((4, 4, DIM, DIM), lambda b: (0, 0, 0, 0)),
            pl.BlockSpec((DIM, 1), lambda b: (0, 0)),
        ],
        out_specs=pl.BlockSpec((1, DIM, N), lambda b: (b, 0, 0)),
        out_shape=jax.ShapeDtypeStruct((B, DIM, N), F32),
    )(xr, wd, bd)


def _k2_body(xd_ref, w_ref, b_ref, out_ref, ca_ref):
    xh = xd_ref[0]
    qkv = _dot(w_ref[...], xh, 1, 0) + b_ref[:, 0][:, None]
    q = qkv[:HD]
    k = qkv[HD:2 * HD]
    v = qkv[2 * HD:]
    s = _dot(k, q, 0, 0)
    m = jnp.max(s, axis=1, keepdims=True)
    e = jnp.exp(s - m)
    l = jnp.sum(e, axis=1, keepdims=True)
    r = 1.0 / l
    ca_ref[0, 0] = _dot(e, r, 0, 0)[:, 0]
    g = _dotb(v, e, 1, 1)
    out_ref[0, 0] = g * r[:, 0][None, :]


def _attn1(xd, W_qkv1, b_qkv1):
    B = xd.shape[0]
    out, ca = pl.pallas_call(
        _k2_body,
        grid=(B, NH),
        in_specs=[
            pl.BlockSpec((1, HD, N), lambda b, h: (b, h, 0)),
            pl.BlockSpec((3 * HD, HD), lambda b, h: (0, 0)),
            pl.BlockSpec((3 * HD, 1), lambda b, h: (0, 0)),
        ],
        out_specs=[
            pl.BlockSpec((1, 1, HD, N), lambda b, h: (b, h, 0, 0)),
            pl.BlockSpec((1, 1, N), lambda b, h: (b * NH + h, 0, 0)),
        ],
        out_shape=[
            jax.ShapeDtypeStruct((B, NH, HD, N), F32),
            jax.ShapeDtypeStruct((B * NH, 1, N), F32),
        ],
    )(xd, W_qkv1, b_qkv1.reshape(3 * HD, 1))
    return out, ca


_CT_TAPS = {0: ((1, 0), (3, -1)), 1: ((2, 0), (0, 1))}


def _k4_body(xu_ref, wu_ref, bu_ref, out_ref):
    for a in range(2):
        for b2 in range(2):
            acc = jnp.zeros((DIM, N), F32)
            for (di, si) in _CT_TAPS[a]:
                for (dj, sj) in _CT_TAPS[b2]:
                    xs = _shift_grid(xu_ref[0], si, sj)
                    acc = acc + _dotb(wu_ref[di, dj], xs, 0, 0)
            out_ref[0, a, b2] = acc + bu_ref[:, 0][:, None]


def _conv_up(xu, W_up, b_up):
    B = xu.shape[0]
    wu = W_up.transpose(2, 3, 0, 1)
    bu = b_up.reshape(DIM, 1)
    return pl.pallas_call(
        _k4_body,
        grid=(B,),
        in_specs=[
            pl.BlockSpec((1, DIM, N), lambda b: (b, 0, 0)),
            pl.BlockSpec((4, 4, DIM, DIM), lambda b: (0, 0, 0, 0)),
            pl.BlockSpec((DIM, 1), lambda b: (0, 0)),
        ],
        out_specs=pl.BlockSpec((1, 2, 2, DIM, N), lambda b: (b, 0, 0, 0, 0)),
        out_shape=jax.ShapeDtypeStruct((B, 2, 2, DIM, N), F32),
    )(xu, wu, bu)


def _k3_body(ca_ref, pos_ref):
    ca_i = lax.bitcast_convert_type(ca_ref[...], jnp.int32)
    p = jnp.zeros((ca_ref.shape[0], 1), jnp.int32)
    for bit in range(30, -1, -1):
        cand = p | jnp.int32(1 << bit)
        cnt = jnp.sum((ca_i >= cand).astype(jnp.int32), axis=1, keepdims=True)
        p = jnp.where(cnt >= KF, cand, p)
    gt = (ca_i > p)
    eq = (ca_i == p)
    m = jnp.sum(gt.astype(jnp.int32), axis=1, keepdims=True)
    need = (KF - m).astype(F32)
    tri = (lax.broadcasted_iota(jnp.int32, (N, N), 0)
           <= lax.broadcasted_iota(jnp.int32, (N, N), 1)).astype(F32)
    cum_eq = _dot(eq.astype(F32), tri, 1, 0)
    sel = jnp.logical_or(gt, jnp.logical_and(eq, cum_eq <= need))
    sel_f = sel.astype(F32)
    cs = _dot(sel_f, tri, 1, 0)
    pos_ref[...] = jnp.where(sel, cs - 1.0, -1.0)


def _topk_pos(ca_all):
    R = ca_all.shape[0]
    return pl.pallas_call(
        _k3_body,
        grid=(1,),
        in_specs=[pl.BlockSpec((R, N), lambda i: (0, 0))],
        out_specs=pl.BlockSpec((R, N), lambda i: (0, 0)),
        out_shape=jax.ShapeDtypeStruct((R, N), F32),
    )(ca_all)


def _k5_body(pos_ref, cph_ref, w_ref, b_ref, scat_ref):
    posv = pos_ref[0]
    rows = lax.broadcasted_iota(jnp.int32, (KF, N), 0).astype(F32)
    smat = jnp.where(jnp.logical_and(rows == posv, posv >= 0.0), 1.0, 0.0)
    toks = []
    for u in range(2):
        for v in range(2):
            toks.append(_dotb(smat, cph_ref[0, u, v], 1, 1))
    tok = jnp.concatenate(toks, axis=0)
    qkv = _dotb(tok, w_ref[...], 1, 1) + b_ref[0]
    q = qkv[:, :HD]
    k = qkv[:, HD:2 * HD]
    v2 = qkv[:, 2 * HD:]
    s = _dotb(k, q, 1, 1)
    mx = jnp.max(s, axis=1, keepdims=True)
    e = jnp.exp(s - mx)
    l = jnp.sum(e, axis=1, keepdims=True)
    out2 = _dotb(e, v2, 1, 0) / l
    for t in range(4):
        u, v = t // 2, t % 2
        o = out2[KF * t:KF * (t + 1)]
        scat_ref[0, 0, u, v] = _dotb(smat, o, 0, 0)


def _topk_attn2(pos, cp, W_qkv2, b_qkv2):
    B = cp.shape[0]
    return pl.pallas_call(
        _k5_body,
        grid=(B, NH),
        in_specs=[
            pl.BlockSpec((1, 1, N), lambda b, h: (b * NH + h, 0, 0)),
            pl.BlockSpec((1, 2, 2, HD, N), lambda b, h: (b, 0, 0, h, 0)),
            pl.BlockSpec((3 * HD, HD), lambda b, h: (0, 0)),
            pl.BlockSpec((1, 3 * HD), lambda b, h: (0, 0)),
        ],
        out_specs=pl.BlockSpec((1, 1, 2, 2, N, HD),
                               lambda b, h: (b, h, 0, 0, 0, 0)),
        out_shape=jax.ShapeDtypeStruct((B, NH, 2, 2, N, HD), F32),
    )(pos, cp, W_qkv2, b_qkv2.reshape(1, 3 * HD))


def _shift_cols(x, k):
    if k > 0:
        return jnp.concatenate(
            [x[:, k:], jnp.zeros((x.shape[0], k), F32)], axis=1)
    if k < 0:
        return jnp.concatenate(
            [jnp.zeros((x.shape[0], -k), F32), x[:, :k]], axis=1)
    return x


def _k6_body(c_sp_ref, c_scr_ref, s_scr_ref, wdw_ref, wpw_ref,
             s1_ref, o1_ref, s2_ref, o2_ref, out_ref):
    y = c_sp_ref[0] + c_scr_ref[0] + s_scr_ref[0]
    colv = lax.broadcasted_iota(jnp.int32, (1, 4096), 1) % 64
    acc = jnp.zeros((DIM, 4096), F32)
    for dy in range(3):
        for dx in range(3):
            t = _shift_cols(y, 64 * (dy - 1) + (dx - 1))
            if dx == 0:
                t = jnp.where(colv >= 1, t, 0.0)
            elif dx == 2:
                t = jnp.where(colv <= 62, t, 0.0)
            acc = acc + wdw_ref[dy * 3 + dx, :][:, None] * t
    t = jnp.clip(acc * s1_ref[:, 0][:, None] + o1_ref[:, 0][:, None],
                 0.0, 6.0)
    z = _dotb(wpw_ref[...], t, 1, 0)
    out_ref[0] = jnp.clip(z * s2_ref[:, 0][:, None] + o2_ref[:, 0][:, None],
                          0.0, 6.0)


def _combine(c_sp, c_scr, s_scr, W_dw, W_pw, g1, b1, m1, v1, g2, b2, m2, v2):
    B = c_sp.shape[0]
    c_sp = c_sp.reshape(B, DIM, 4096)
    c_scr = c_scr.reshape(B, DIM, 4096)
    s_scr = s_scr.reshape(B, DIM, 4096)
    inv1 = g1 / jnp.sqrt(v1 + 1e-5)
    inv2 = g2 / jnp.sqrt(v2 + 1e-5)
    s1 = inv1.reshape(DIM, 1)
    o1 = (b1 - m1 * inv1).reshape(DIM, 1)
    s2 = inv2.reshape(DIM, 1)
    o2 = (b2 - m2 * inv2).reshape(DIM, 1)
    wdw = W_dw.reshape(DIM, 9).T.reshape(9, DIM)
    wpw = W_pw.reshape(DIM, DIM)
    full = lambda shape: pl.BlockSpec(shape, lambda b: (0,) * len(shape))
    return pl.pallas_call(
        _k6_body,
        grid=(B,),
        in_specs=[
            pl.BlockSpec((1, DIM, 4096), lambda b: (b, 0, 0)),
            pl.BlockSpec((1, DIM, 4096), lambda b: (b, 0, 0)),
            pl.BlockSpec((1, DIM, 4096), lambda b: (b, 0, 0)),
            full((9, DIM)),
            full((DIM, DIM)),
            full((DIM, 1)), full((DIM, 1)), full((DIM, 1)), full((DIM, 1)),
        ],
        out_specs=pl.BlockSpec((1, DIM, 4096), lambda b: (b, 0, 0)),
        out_shape=jax.ShapeDtypeStruct((B, DIM, 4096), F32),
    )(c_sp, c_scr, s_scr, wdw, wpw, s1, o1, s2, o2)


def kernel(x, W_down, b_down, W_up, b_up, W_qkv1, b_qkv1, W_qkv2, b_qkv2,
           W_dw, W_pw, gamma1, beta1, mean1, var1, gamma2, beta2, mean2,
           var2):
    B = x.shape[0]
    xd = _conv_down(x, W_down, b_down)
    out1, ca = _attn1(xd, W_qkv1, b_qkv1)
    xu = out1.reshape(B, DIM, N)
    cp = _conv_up(xu, W_up, b_up)
    pos = _topk_pos(ca.reshape(B * NH, N)).reshape(B * NH, 1, N)
    scat = _topk_attn2(pos, cp, W_qkv2, b_qkv2)
    c_sp = cp.reshape(B, 2, 2, DIM, GRID, GRID).transpose(
        0, 3, 4, 1, 5, 2).reshape(B, DIM, 64, 64)
    c_scr = cp.reshape(B, 2, 2, DIM, GRID, 2, 16).transpose(
        0, 3, 4, 5, 6, 1, 2).reshape(B, DIM, 64, 64)
    s_scr = scat.reshape(B, NH, 2, 2, GRID, 2, 16, HD).transpose(
        0, 1, 7, 4, 5, 6, 2, 3).reshape(B, DIM, 64, 64)
    y = _combine(c_sp, c_scr, s_scr, W_dw, W_pw, gamma1, beta1, mean1,
                 var1, gamma2, beta2, mean2, var2)
    return y.reshape(B, DIM, 64, 64)


# speedup vs baseline: 2.0258x; 1.0010x over previous
import jax
import jax.numpy as jnp
from jax import lax
from jax.experimental import pallas as pl

DIM = 256
HD = 64
NH = 4
GRID = 32
N = GRID * GRID
KF = 256
F32 = jnp.float32


def _dot(a, b, ca, cb):
    return lax.dot_general(a, b, (((ca,), (cb,)), ((), ())),
                           preferred_element_type=F32)


def _dotb(a, b, ca, cb):
    return lax.dot_general(a.astype(jnp.bfloat16), b.astype(jnp.bfloat16),
                           (((ca,), (cb,)), ((), ())),
                           preferred_element_type=F32)


def _shift_grid(x, si, sj):
    t = _shift_cols(x, GRID * si + sj)
    colv = lax.broadcasted_iota(jnp.int32, (1, N), 1) % GRID
    if sj == -1:
        t = jnp.where(colv >= 1, t, 0.0)
    elif sj == 1:
        t = jnp.where(colv <= GRID - 2, t, 0.0)
    return t


def _k1_body(xr_ref, wd_ref, bd_ref, out_ref):
    acc = jnp.zeros((DIM, N), F32)
    for di in range(4):
        pa = (di - 1) % 2
        si = (di - 1) // 2
        for dj in range(4):
            pb = (dj - 1) % 2
            sj = (dj - 1) // 2
            xs = _shift_grid(xr_ref[0, pa, pb], si, sj)
            acc = acc + _dot(wd_ref[di, dj], xs, 1, 0)
    out_ref[0] = acc + bd_ref[:, 0][:, None]


def _conv_down(x, W_down, b_down):
    B = x.shape[0]
    xr = x.reshape(B, DIM, GRID, 2, GRID, 2).transpose(
        0, 3, 5, 1, 2, 4).reshape(B, 2, 2, DIM, N)
    wd = W_down.transpose(2, 3, 0, 1)
    bd = b_down.reshape(DIM, 1)
    return pl.pallas_call(
        _k1_body,
        grid=(B,),
        in_specs=[
            pl.BlockSpec((1, 2, 2, DIM, N), lambda b: (b, 0, 0, 0, 0)),
            pl.BlockSpec((4, 4, DIM, DIM), lambda b: (0, 0, 0, 0)),
            pl.BlockSpec((DIM, 1), lambda b: (0, 0)),
        ],
        out_specs=pl.BlockSpec((1, DIM, N), lambda b: (b, 0, 0)),
        out_shape=jax.ShapeDtypeStruct((B, DIM, N), F32),
    )(xr, wd, bd)


def _k2_body(xd_ref, w_ref, b_ref, out_ref, ca_ref):
    xh = xd_ref[0]
    qkv = _dot(w_ref[...], xh, 1, 0) + b_ref[:, 0][:, None]
    q = qkv[:HD]
    k = qkv[HD:2 * HD]
    v = qkv[2 * HD:]
    s = _dot(k, q, 0, 0)
    m = jnp.max(s, axis=1, keepdims=True)
    e = jnp.exp(s - m)
    l = jnp.sum(e, axis=1, keepdims=True)
    r = 1.0 / l
    ca_ref[0, 0] = _dot(e, r, 0, 0)[:, 0]
    g = _dotb(v, e, 1, 1)
    out_ref[0, 0] = g * r[:, 0][None, :]


def _attn1(xd, W_qkv1, b_qkv1):
    B = xd.shape[0]
    out, ca = pl.pallas_call(
        _k2_body,
        grid=(B, NH),
        in_specs=[
            pl.BlockSpec((1, HD, N), lambda b, h: (b, h, 0)),
            pl.BlockSpec((3 * HD, HD), lambda b, h: (0, 0)),
            pl.BlockSpec((3 * HD, 1), lambda b, h: (0, 0)),
        ],
        out_specs=[
            pl.BlockSpec((1, 1, HD, N), lambda b, h: (b, h, 0, 0)),
            pl.BlockSpec((1, 1, N), lambda b, h: (b * NH + h, 0, 0)),
        ],
        out_shape=[
            jax.ShapeDtypeStruct((B, NH, HD, N), F32),
            jax.ShapeDtypeStruct((B * NH, 1, N), F32),
        ],
    )(xd, W_qkv1, b_qkv1.reshape(3 * HD, 1))
    return out, ca


_CT_TAPS = {0: ((1, 0), (3, -1)), 1: ((2, 0), (0, 1))}


def _k4_body(xu_ref, wu_ref, bu_ref, out_ref):
    for a in range(2):
        for b2 in range(2):
            acc = jnp.zeros((DIM, N), F32)
            for (di, si) in _CT_TAPS[a]:
                for (dj, sj) in _CT_TAPS[b2]:
                    xs = _shift_grid(xu_ref[0], si, sj)
                    acc = acc + _dotb(wu_ref[di, dj], xs, 0, 0)
            out_ref[0, a, b2] = acc + bu_ref[:, 0][:, None]


def _conv_up(xu, W_up, b_up):
    B = xu.shape[0]
    wu = W_up.transpose(2, 3, 0, 1)
    bu = b_up.reshape(DIM, 1)
    return pl.pallas_call(
        _k4_body,
        grid=(B,),
        in_specs=[
            pl.BlockSpec((1, DIM, N), lambda b: (b, 0, 0)),
            pl.BlockSpec((4, 4, DIM, DIM), lambda b: (0, 0, 0, 0)),
            pl.BlockSpec((DIM, 1), lambda b: (0, 0)),
        ],
        out_specs=pl.BlockSpec((1, 2, 2, DIM, N), lambda b: (b, 0, 0, 0, 0)),
        out_shape=jax.ShapeDtypeStruct((B, 2, 2, DIM, N), F32),
    )(xu, wu, bu)


def _k3_body(ca_ref, pos_ref):
    ca_i = lax.bitcast_convert_type(ca_ref[...], jnp.int32)
    p = jnp.zeros((ca_ref.shape[0], 1), jnp.int32)
    for bit in range(30, -1, -1):
        cand = p | jnp.int32(1 << bit)
        cnt = jnp.sum((ca_i >= cand).astype(jnp.int32), axis=1, keepdims=True)
        p = jnp.where(cnt >= KF, cand, p)
    gt = (ca_i > p)
    eq = (ca_i == p)
    m = jnp.sum(gt.astype(jnp.int32), axis=1, keepdims=True)
    need = (KF - m).astype(F32)
    tri = (lax.broadcasted_iota(jnp.int32, (N, N), 0)
           <= lax.broadcasted_iota(jnp.int32, (N, N), 1)).astype(F32)
    cum_eq = _dot(eq.astype(F32), tri, 1, 0)
    sel = jnp.logical_or(gt, jnp.logical_and(eq, cum_eq <= need))
    sel_f = sel.astype(F32)
    cs = _dot(sel_f, tri, 1, 0)
    pos_ref[...] = jnp.where(sel, cs - 1.0, -1.0)


def _topk_pos(ca_all):
    R = ca_all.shape[0]
    return pl.pallas_call(
        _k3_body,
        grid=(1,),
        in_specs=[pl.BlockSpec((R, N), lambda i: (0, 0))],
        out_specs=pl.BlockSpec((R, N), lambda i: (0, 0)),
        out_shape=jax.ShapeDtypeStruct((R, N), F32),
    )(ca_all)


def _k5_body(pos_ref, cph_ref, w_ref, b_ref, scat_ref):
    posv = pos_ref[0]
    rows = lax.broadcasted_iota(jnp.int32, (KF, N), 0).astype(F32)
    smat = jnp.where(jnp.logical_and(rows == posv, posv >= 0.0), 1.0, 0.0)
    toks = []
    for u in range(2):
        for v in range(2):
            toks.append(_dotb(smat, cph_ref[0, u, v], 1, 1))
    tok = jnp.concatenate(toks, axis=0)
    qkv = _dotb(tok, w_ref[...], 1, 1) + b_ref[0]
    q = qkv[:, :HD]
    k = qkv[:, HD:2 * HD]
    v2 = qkv[:, 2 * HD:]
    s = _dotb(k, q, 1, 1)
    mx = jnp.max(s, axis=1, keepdims=True)
    e = jnp.exp(s - mx)
    l = jnp.sum(e, axis=1, keepdims=True)
    out2 = _dotb(e, v2, 1, 0) / l
    for t in range(4):
        u, v = t // 2, t % 2
        o = out2[KF * t:KF * (t + 1)]
        scat_ref[0, 0, u, v] = _dotb(smat, o, 0, 0)


def _topk_attn2(pos, cp, W_qkv2, b_qkv2):
    B = cp.shape[0]
    return pl.pallas_call(
        _k5_body,
        grid=(B, NH),
        in_specs=[
            pl.BlockSpec((1, 1, N), lambda b, h: (b * NH + h, 0, 0)),
            pl.BlockSpec((1, 2, 2, HD, N), lambda b, h: (b, 0, 0, h, 0)),
            pl.BlockSpec((3 * HD, HD), lambda b, h: (0, 0)),
            pl.BlockSpec((1, 3 * HD), lambda b, h: (0, 0)),
        ],
        out_specs=pl.BlockSpec((1, 1, 2, 2, N, HD),
                               lambda b, h: (b, h, 0, 0, 0, 0)),
        out_shape=jax.ShapeDtypeStruct((B, NH, 2, 2, N, HD), F32),
    )(pos, cp, W_qkv2, b_qkv2.reshape(1, 3 * HD))


def _shift_cols(x, k):
    if k > 0:
        return jnp.concatenate(
            [x[:, k:], jnp.zeros((x.shape[0], k), F32)], axis=1)
    if k < 0:
        return jnp.concatenate(
            [jnp.zeros((x.shape[0], -k), F32), x[:, :k]], axis=1)
    return x


def _k6_body(c_sp_ref, c_scr_ref, s_scr_ref, wdw_ref, wpw_ref,
             s1_ref, o1_ref, s2_ref, o2_ref, out_ref):
    y = c_sp_ref[0] + c_scr_ref[0] + s_scr_ref[0]
    colv = lax.broadcasted_iota(jnp.int32, (1, 4096), 1) % 64
    acc = jnp.zeros((DIM, 4096), F32)
    for dy in range(3):
        for dx in range(3):
            t = _shift_cols(y, 64 * (dy - 1) + (dx - 1))
            if dx == 0:
                t = jnp.where(colv >= 1, t, 0.0)
            elif dx == 2:
                t = jnp.where(colv <= 62, t, 0.0)
            acc = acc + wdw_ref[dy * 3 + dx, :][:, None] * t
    t = jnp.clip(acc * s1_ref[:, 0][:, None] + o1_ref[:, 0][:, None],
                 0.0, 6.0)
    z = _dotb(wpw_ref[...], t, 1, 0)
    out_ref[0] = jnp.clip(z * s2_ref[:, 0][:, None] + o2_ref[:, 0][:, None],
                          0.0, 6.0)


def _combine(c_sp, c_scr, s_scr, W_dw, W_pw, g1, b1, m1, v1, g2, b2, m2, v2):
    B = c_sp.shape[0]
    c_sp = c_sp.reshape(B, DIM, 4096)
    c_scr = c_scr.reshape(B, DIM, 4096)
    s_scr = s_scr.reshape(B, DIM, 4096)
    inv1 = g1 / jnp.sqrt(v1 + 1e-5)
    inv2 = g2 / jnp.sqrt(v2 + 1e-5)
    s1 = inv1.reshape(DIM, 1)
    o1 = (b1 - m1 * inv1).reshape(DIM, 1)
    s2 = inv2.reshape(DIM, 1)
    o2 = (b2 - m2 * inv2).reshape(DIM, 1)
    wdw = W_dw.reshape(DIM, 9).T.reshape(9, DIM)
    wpw = W_pw.reshape(DIM, DIM)
    full = lambda shape: pl.BlockSpec(shape, lambda b: (0,) * len(shape))
    return pl.pallas_call(
        _k6_body,
        grid=(B,),
        in_specs=[
            pl.BlockSpec((1, DIM, 4096), lambda b: (b, 0, 0)),
            pl.BlockSpec((1, DIM, 4096), lambda b: (b, 0, 0)),
            pl.BlockSpec((1, DIM, 4096), lambda b: (b, 0, 0)),
            full((9, DIM)),
            full((DIM, DIM)),
            full((DIM, 1)), full((DIM, 1)), full((DIM, 1)), full((DIM, 1)),
        ],
        out_specs=pl.BlockSpec((1, DIM, 4096), lambda b: (b, 0, 0)),
        out_shape=jax.ShapeDtypeStruct((B, DIM, 4096), F32),
    )(c_sp, c_scr, s_scr, wdw, wpw, s1, o1, s2, o2)


def kernel(x, W_down, b_down, W_up, b_up, W_qkv1, b_qkv1, W_qkv2, b_qkv2,
           W_dw, W_pw, gamma1, beta1, mean1, var1, gamma2, beta2, mean2,
           var2):
    B = x.shape[0]
    xd = _conv_down(x, W_down, b_down)
    out1, ca = _attn1(xd, W_qkv1, b_qkv1)
    xu = out1.reshape(B, DIM, N)
    cp = _conv_up(xu, W_up, b_up)
    pos = _topk_pos(ca.reshape(B * NH, N)).reshape(B * NH, 1, N)
    scat = _topk_attn2(pos, cp, W_qkv2, b_qkv2)
    c_sp = cp.reshape(B, 2, 2, DIM, GRID, GRID).transpose(
        0, 3, 4, 1, 5, 2).reshape(B, DIM, 64, 64)
    c_scr = cp.reshape(B, 2, 2, DIM, GRID, 2, 16).transpose(
        0, 3, 4, 5, 6, 1, 2).reshape(B, DIM, 64, 64)
    s_scr = scat.reshape(B, NH, 2, 2, GRID, 2, 16, HD).transpose(
        0, 1, 7, 4, 5, 6, 2, 3).reshape(B, DIM, 64, 64)
    y = _combine(c_sp, c_scr, s_scr, W_dw, W_pw, gamma1, beta1, mean1,
                 var1, gamma2, beta2, mean2, var2)
    return y.reshape(B, DIM, 64, 64)
